# baseline (speedup 1.0000x reference)
import jax
import jax.numpy as jnp
from jax.experimental import pallas as pl


def _copy_kernel(in_ref, out_ref):
    out_ref[...] = in_ref[...]


def kernel(xyz, reconstruction, neighborsMatrix, numNeighbors, weightMatrix, arapWeight):
    n, d = reconstruction.shape
    blk = 25000
    return pl.pallas_call(
        _copy_kernel,
        grid=(pl.cdiv(n, blk),),
        in_specs=[pl.BlockSpec((blk, d), lambda i: (i, 0))],
        out_specs=pl.BlockSpec((blk, d), lambda i: (i, 0)),
        out_shape=jax.ShapeDtypeStruct(reconstruction.shape, reconstruction.dtype),
    )(reconstruction)
